# baseline (speedup 1.0000x reference)
import functools

import jax
import jax.numpy as jnp
from jax import lax
from jax.experimental import pallas as pl
from jax.experimental.pallas import tpu as pltpu
from jax.experimental.pallas import tpu_sc as plsc

_LANES = 16


def _sc_geometry():
    try:
        info = plsc.get_sparse_core_info()
        return info.num_cores, info.num_subcores
    except Exception:
        return 2, 16


def _make_kernel(n, num_types):
    nc, ns = _sc_geometry()
    nw = nc * ns
    assert n % nw == 0
    per_w = n // nw
    chunk = per_w
    for cand in (25600, 12800, 6400, 3200, 1600, 800, 400, 80, 16):
        if per_w % cand == 0:
            chunk = cand
            break
    n_chunks = per_w // chunk

    mesh = plsc.VectorSubcoreMesh(core_axis_name="c", subcore_axis_name="s")

    @functools.partial(
        pl.kernel,
        mesh=mesh,
        out_type=jax.ShapeDtypeStruct((n,), jnp.float32),
        compiler_params=pltpu.CompilerParams(needs_layout_passes=False),
        scratch_types=[
            pltpu.VMEM((max(num_types, 128),), jnp.float32),
            pltpu.VMEM((chunk,), jnp.int32),
            pltpu.VMEM((chunk,), jnp.float32),
        ],
    )
    def run(num_hbm, eng_hbm, tab_hbm, out_hbm, tab_v, num_v, eng_v):
        wid = lax.axis_index("s") * nc + lax.axis_index("c")
        base = wid * per_w
        pltpu.sync_copy(tab_hbm, tab_v.at[pl.ds(0, num_types)])

        def do_chunk(c, carry):
            off = base + c * chunk
            pltpu.sync_copy(num_hbm.at[pl.ds(off, chunk)], num_v)
            pltpu.sync_copy(eng_hbm.at[pl.ds(off, chunk)], eng_v)

            def inner(i, carry2):
                s = pl.ds(i * _LANES, _LANES)
                idx = num_v[s]
                vals = plsc.load_gather(tab_v, [idx])
                eng_v[s] = eng_v[s] + vals
                return carry2

            lax.fori_loop(0, chunk // _LANES, inner, 0, unroll=4)
            pltpu.sync_copy(eng_v, out_hbm.at[pl.ds(off, chunk)])
            return carry

        lax.fori_loop(0, n_chunks, do_chunk, 0)

    return run


def kernel(numbers, energy, shifts_weight):
    b, l = energy.shape
    n = b * l
    num_flat = numbers.reshape(n).astype(jnp.int32)
    eng_flat = energy.reshape(n)
    tab_flat = shifts_weight.reshape(-1)
    out = _make_kernel(n, tab_flat.shape[0])(num_flat, eng_flat, tab_flat)
    return out.reshape(b, l)


# speedup vs baseline: 1.4477x; 1.4477x over previous
import functools

import jax
import jax.numpy as jnp
from jax import lax
from jax.experimental import pallas as pl
from jax.experimental.pallas import tpu as pltpu
from jax.experimental.pallas import tpu_sc as plsc

_LANES = 16


def _sc_geometry():
    try:
        info = plsc.get_sparse_core_info()
        return info.num_cores, info.num_subcores
    except Exception:
        return 2, 16


def _make_kernel(n, num_types):
    nc, ns = _sc_geometry()
    nw = nc * ns
    assert n % nw == 0
    per_w = n // nw
    chunk = None
    for cand in (12800, 6400, 3200, 1600, 800, 400, 80, 16):
        if per_w % cand == 0 and (per_w // cand) % 2 == 0:
            chunk = cand
            break
    assert chunk is not None
    n_chunks = per_w // chunk

    mesh = plsc.VectorSubcoreMesh(core_axis_name="c", subcore_axis_name="s")

    @functools.partial(
        pl.kernel,
        mesh=mesh,
        out_type=jax.ShapeDtypeStruct((n,), jnp.float32),
        compiler_params=pltpu.CompilerParams(needs_layout_passes=False),
        scratch_types=[
            pltpu.VMEM((max(num_types, 128),), jnp.float32),
            pltpu.VMEM((chunk,), jnp.int32),
            pltpu.VMEM((chunk,), jnp.int32),
            pltpu.VMEM((chunk,), jnp.float32),
            pltpu.VMEM((chunk,), jnp.float32),
            pltpu.VMEM((chunk,), jnp.float32),
            pltpu.VMEM((chunk,), jnp.float32),
            pltpu.SemaphoreType.DMA,
            pltpu.SemaphoreType.DMA,
            pltpu.SemaphoreType.DMA,
            pltpu.SemaphoreType.DMA,
            pltpu.SemaphoreType.DMA,
            pltpu.SemaphoreType.DMA,
        ],
    )
    def run(num_hbm, eng_hbm, tab_hbm, out_hbm, tab_v,
            num0, num1, eng0, eng1, out0, out1,
            ns0, ns1, es0, es1, ss0, ss1):
        nums, engs, outs = (num0, num1), (eng0, eng1), (out0, out1)
        nsem, esem, ssem = (ns0, ns1), (es0, es1), (ss0, ss1)
        wid = lax.axis_index("s") * nc + lax.axis_index("c")
        base = wid * per_w
        pltpu.sync_copy(tab_hbm, tab_v.at[pl.ds(0, num_types)])

        def issue_loads(c, b):
            off = base + c * chunk
            pltpu.async_copy(num_hbm.at[pl.ds(off, chunk)], nums[b], nsem[b])
            pltpu.async_copy(eng_hbm.at[pl.ds(off, chunk)], engs[b], esem[b])

        issue_loads(0, 0)

        def outer(g, carry):
            for b in range(2):
                c = 2 * g + b
                off = base + c * chunk

                @pl.when(c + 1 < n_chunks)
                def _():
                    issue_loads(c + 1, 1 - b)

                pltpu.make_async_copy(
                    num_hbm.at[pl.ds(off, chunk)], nums[b], nsem[b]).wait()
                pltpu.make_async_copy(
                    eng_hbm.at[pl.ds(off, chunk)], engs[b], esem[b]).wait()

                @pl.when(c >= 2)
                def _():
                    pltpu.make_async_copy(
                        outs[b], out_hbm.at[pl.ds(off, chunk)], ssem[b]).wait()

                nv, ev, ov = nums[b], engs[b], outs[b]

                @plsc.parallel_loop(0, chunk, step=_LANES, unroll=8)
                def body(i):
                    s = pl.ds(i, _LANES)
                    vals = plsc.load_gather(tab_v, [nv[s]])
                    ov[s] = ev[s] + vals

                pltpu.async_copy(ov, out_hbm.at[pl.ds(off, chunk)], ssem[b])
            return carry

        lax.fori_loop(0, n_chunks // 2, outer, 0)
        for b in range(2):
            pltpu.make_async_copy(
                outs[b], out_hbm.at[pl.ds(0, chunk)], ssem[b]).wait()

    return run


def kernel(numbers, energy, shifts_weight):
    b, l = energy.shape
    n = b * l
    num_flat = numbers.reshape(n).astype(jnp.int32)
    eng_flat = energy.reshape(n)
    tab_flat = shifts_weight.reshape(-1)
    out = _make_kernel(n, tab_flat.shape[0])(num_flat, eng_flat, tab_flat)
    return out.reshape(b, l)


# speedup vs baseline: 2.5115x; 1.7348x over previous
import functools

import jax
import jax.numpy as jnp
from jax import lax
from jax.experimental import pallas as pl
from jax.experimental.pallas import tpu as pltpu
from jax.experimental.pallas import tpu_sc as plsc

_LANES = 16
_SUBLANES = 8


def _sc_geometry():
    try:
        info = plsc.get_sparse_core_info()
        return info.num_cores, info.num_subcores
    except Exception:
        return 2, 16


def _col_group_starts(l_cols):
    starts = list(range(0, l_cols - _LANES + 1, _LANES))
    if l_cols % _LANES:
        starts.append(l_cols - _LANES)
    for s in starts:
        assert s % 128 <= 128 - _LANES
    return starts


def _make_kernel(b_rows, l_cols, num_types):
    nc, ns = _sc_geometry()
    nw = nc * ns
    assert b_rows % (nw * _SUBLANES) == 0
    rows_per_w = b_rows // nw
    rc = None
    for cand in (64, 32, 16, 8):
        if rows_per_w % cand == 0 and (rows_per_w // cand) % 2 == 0:
            rc = cand
            break
    assert rc is not None
    n_chunks = rows_per_w // rc
    col_starts = _col_group_starts(l_cols)

    mesh = plsc.VectorSubcoreMesh(core_axis_name="c", subcore_axis_name="s")

    @functools.partial(
        pl.kernel,
        mesh=mesh,
        out_type=jax.ShapeDtypeStruct((b_rows, l_cols), jnp.float32),
        compiler_params=pltpu.CompilerParams(
            needs_layout_passes=False, use_tc_tiling_on_sc=True),
        scratch_types=[
            pltpu.VMEM((max(num_types, 128),), jnp.float32),
            pltpu.VMEM((rc, l_cols), jnp.int32),
            pltpu.VMEM((rc, l_cols), jnp.int32),
            pltpu.VMEM((rc, l_cols), jnp.float32),
            pltpu.VMEM((rc, l_cols), jnp.float32),
            pltpu.VMEM((rc, l_cols), jnp.float32),
            pltpu.VMEM((rc, l_cols), jnp.float32),
            pltpu.SemaphoreType.DMA,
            pltpu.SemaphoreType.DMA,
            pltpu.SemaphoreType.DMA,
            pltpu.SemaphoreType.DMA,
            pltpu.SemaphoreType.DMA,
            pltpu.SemaphoreType.DMA,
        ],
    )
    def run(num_hbm, eng_hbm, tab_hbm, out_hbm, tab_v,
            num0, num1, eng0, eng1, out0, out1,
            ns0, ns1, es0, es1, ss0, ss1):
        nums, engs, outs = (num0, num1), (eng0, eng1), (out0, out1)
        nsem, esem, ssem = (ns0, ns1), (es0, es1), (ss0, ss1)
        wid = lax.axis_index("s") * nc + lax.axis_index("c")
        base = wid * rows_per_w
        pltpu.sync_copy(tab_hbm, tab_v.at[pl.ds(0, num_types)])

        def issue_loads(c, b):
            row0 = base + c * rc
            pltpu.async_copy(num_hbm.at[pl.ds(row0, rc)], nums[b], nsem[b])
            pltpu.async_copy(eng_hbm.at[pl.ds(row0, rc)], engs[b], esem[b])

        issue_loads(0, 0)

        def outer(g, carry):
            for b in range(2):
                c = 2 * g + b
                row0 = base + c * rc

                @pl.when(c + 1 < n_chunks)
                def _():
                    issue_loads(c + 1, 1 - b)

                pltpu.make_async_copy(
                    num_hbm.at[pl.ds(row0, rc)], nums[b], nsem[b]).wait()
                pltpu.make_async_copy(
                    eng_hbm.at[pl.ds(row0, rc)], engs[b], esem[b]).wait()

                @pl.when(c >= 2)
                def _():
                    pltpu.make_async_copy(
                        outs[b], out_hbm.at[pl.ds(row0, rc)], ssem[b]).wait()

                nv, ev, ov = nums[b], engs[b], outs[b]

                @plsc.parallel_loop(0, rc, step=1, unroll=2)
                def body(r):
                    for cs in col_starts:
                        s = (r, pl.ds(cs, _LANES))
                        vals = plsc.load_gather(tab_v, [nv[s]])
                        ov[s] = ev[s] + vals

                pltpu.async_copy(ov, out_hbm.at[pl.ds(row0, rc)], ssem[b])
            return carry

        lax.fori_loop(0, n_chunks // 2, outer, 0)
        for b in range(2):
            pltpu.make_async_copy(
                outs[b], out_hbm.at[pl.ds(0, rc)], ssem[b]).wait()

    return run


def kernel(numbers, energy, shifts_weight):
    b, l = energy.shape
    tab_flat = shifts_weight.reshape(-1)
    out = _make_kernel(b, l, tab_flat.shape[0])(
        numbers.astype(jnp.int32), energy, tab_flat)
    return out


# speedup vs baseline: 4.7577x; 1.8944x over previous
import functools

import jax
import jax.numpy as jnp
from jax import lax
from jax.experimental import pallas as pl
from jax.experimental.pallas import tpu as pltpu
from jax.experimental.pallas import tpu_sc as plsc

_LANES = 16
_SUBLANES = 8


def _sc_geometry():
    try:
        info = plsc.get_sparse_core_info()
        return info.num_cores, info.num_subcores
    except Exception:
        return 2, 16


def _make_kernel(rows, cols, num_types):
    nc, ns = _sc_geometry()
    nw = nc * ns
    assert rows % _SUBLANES == 0 and cols % (nw * 128) == 0
    lanes_per_w = cols // nw
    n_chunks = rows // _SUBLANES
    assert n_chunks >= 3

    mesh = plsc.VectorSubcoreMesh(core_axis_name="c", subcore_axis_name="s")

    @functools.partial(
        pl.kernel,
        mesh=mesh,
        out_type=jax.ShapeDtypeStruct((rows, cols), jnp.float32),
        compiler_params=pltpu.CompilerParams(
            needs_layout_passes=False, use_tc_tiling_on_sc=True),
        scratch_types=[
            pltpu.VMEM((max(num_types, 128),), jnp.float32),
            pltpu.VMEM((_SUBLANES, lanes_per_w), jnp.int32),
            pltpu.VMEM((_SUBLANES, lanes_per_w), jnp.int32),
            pltpu.VMEM((_SUBLANES, lanes_per_w), jnp.float32),
            pltpu.VMEM((_SUBLANES, lanes_per_w), jnp.float32),
            pltpu.VMEM((_SUBLANES, lanes_per_w), jnp.float32),
            pltpu.VMEM((_SUBLANES, lanes_per_w), jnp.float32),
            pltpu.SemaphoreType.DMA,
            pltpu.SemaphoreType.DMA,
            pltpu.SemaphoreType.DMA,
            pltpu.SemaphoreType.DMA,
            pltpu.SemaphoreType.DMA,
            pltpu.SemaphoreType.DMA,
        ],
    )
    def run(num_hbm, eng_hbm, tab_hbm, out_hbm, tab_v,
            num0, num1, eng0, eng1, out0, out1,
            ns0, ns1, es0, es1, ss0, ss1):
        nums, engs, outs = (num0, num1), (eng0, eng1), (out0, out1)
        nsem, esem, ssem = (ns0, ns1), (es0, es1), (ss0, ss1)
        wid = lax.axis_index("s") * nc + lax.axis_index("c")
        lane0 = wid * lanes_per_w
        pltpu.sync_copy(tab_hbm, tab_v.at[pl.ds(0, num_types)])

        def hslice(c):
            return (pl.ds(c * _SUBLANES, _SUBLANES), pl.ds(lane0, lanes_per_w))

        def issue_loads(c, b):
            s = hslice(c)
            pltpu.async_copy(num_hbm.at[s], nums[b], nsem[b])
            pltpu.async_copy(eng_hbm.at[s], engs[b], esem[b])

        def wait_loads(c, b):
            s = hslice(c)
            pltpu.make_async_copy(num_hbm.at[s], nums[b], nsem[b]).wait()
            pltpu.make_async_copy(eng_hbm.at[s], engs[b], esem[b]).wait()

        def wait_store(c, b):
            pltpu.make_async_copy(outs[b], out_hbm.at[hslice(c)], ssem[b]).wait()

        def compute(b):
            nv, ev, ov = nums[b], engs[b], outs[b]

            @plsc.parallel_loop(0, lanes_per_w, step=_LANES, unroll=2)
            def body(cs):
                for r in range(_SUBLANES):
                    s = (r, pl.ds(cs, _LANES))
                    vals = plsc.load_gather(tab_v, [nv[s]])
                    ov[s] = ev[s] + vals

        issue_loads(0, 0)

        def pair(g, carry):
            for b in range(2):
                c = 2 * g + b

                @pl.when(c + 1 < n_chunks)
                def _():
                    issue_loads(c + 1, 1 - b)

                wait_loads(c, b)

                @pl.when(c >= 2)
                def _():
                    wait_store(c - 2, b)

                compute(b)
                pltpu.async_copy(outs[b], out_hbm.at[hslice(c)], ssem[b])
            return carry

        lax.fori_loop(0, n_chunks // 2, pair, 0)
        if n_chunks % 2:
            c = n_chunks - 1
            wait_loads(c, 0)
            wait_store(c - 2, 0)
            compute(0)
            pltpu.async_copy(outs[0], out_hbm.at[hslice(c)], ssem[0])
        wait_store(n_chunks - 1, (n_chunks - 1) % 2)
        wait_store(n_chunks - 2, (n_chunks - 2) % 2)

    return run


def kernel(numbers, energy, shifts_weight):
    b, l = energy.shape
    tab_flat = shifts_weight.reshape(-1)
    out_t = _make_kernel(l, b, tab_flat.shape[0])(
        numbers.astype(jnp.int32).T, energy.T, tab_flat)
    return out_t.T


# speedup vs baseline: 4.7787x; 1.0044x over previous
import functools

import jax
import jax.numpy as jnp
from jax import lax
from jax.experimental import pallas as pl
from jax.experimental.pallas import tpu as pltpu
from jax.experimental.pallas import tpu_sc as plsc

_LANES = 16
_SUBLANES = 8


def _sc_geometry():
    try:
        info = plsc.get_sparse_core_info()
        return info.num_cores, info.num_subcores
    except Exception:
        return 2, 16


def _make_kernel(rows, cols, num_types):
    nc, ns = _sc_geometry()
    nw = nc * ns
    assert rows % _SUBLANES == 0 and cols % (nw * 128) == 0
    lanes_per_w = cols // nw
    chunk_rows = None
    for cand in (40, 24, 16, 8):
        if rows % cand == 0 and 6 * cand * lanes_per_w * 4 <= 490_000:
            chunk_rows = cand
            break
    assert chunk_rows is not None and chunk_rows % _SUBLANES == 0
    n_chunks = rows // chunk_rows
    assert n_chunks >= 3

    mesh = plsc.VectorSubcoreMesh(core_axis_name="c", subcore_axis_name="s")

    @functools.partial(
        pl.kernel,
        mesh=mesh,
        out_type=jax.ShapeDtypeStruct((rows, cols), jnp.float32),
        compiler_params=pltpu.CompilerParams(
            needs_layout_passes=False, use_tc_tiling_on_sc=True),
        scratch_types=[
            pltpu.VMEM((max(num_types, 128),), jnp.float32),
            pltpu.VMEM((chunk_rows, lanes_per_w), jnp.int32),
            pltpu.VMEM((chunk_rows, lanes_per_w), jnp.int32),
            pltpu.VMEM((chunk_rows, lanes_per_w), jnp.float32),
            pltpu.VMEM((chunk_rows, lanes_per_w), jnp.float32),
            pltpu.VMEM((chunk_rows, lanes_per_w), jnp.float32),
            pltpu.VMEM((chunk_rows, lanes_per_w), jnp.float32),
            pltpu.SemaphoreType.DMA,
            pltpu.SemaphoreType.DMA,
            pltpu.SemaphoreType.DMA,
            pltpu.SemaphoreType.DMA,
            pltpu.SemaphoreType.DMA,
            pltpu.SemaphoreType.DMA,
        ],
    )
    def run(num_hbm, eng_hbm, tab_hbm, out_hbm, tab_v,
            num0, num1, eng0, eng1, out0, out1,
            ns0, ns1, es0, es1, ss0, ss1):
        nums, engs, outs = (num0, num1), (eng0, eng1), (out0, out1)
        nsem, esem, ssem = (ns0, ns1), (es0, es1), (ss0, ss1)
        wid = lax.axis_index("s") * nc + lax.axis_index("c")
        lane0 = wid * lanes_per_w
        pltpu.sync_copy(tab_hbm, tab_v.at[pl.ds(0, num_types)])

        def hslice(c):
            return (pl.ds(c * chunk_rows, chunk_rows), pl.ds(lane0, lanes_per_w))

        def issue_loads(c, b):
            s = hslice(c)
            pltpu.async_copy(num_hbm.at[s], nums[b], nsem[b])
            pltpu.async_copy(eng_hbm.at[s], engs[b], esem[b])

        def wait_loads(c, b):
            s = hslice(c)
            pltpu.make_async_copy(num_hbm.at[s], nums[b], nsem[b]).wait()
            pltpu.make_async_copy(eng_hbm.at[s], engs[b], esem[b]).wait()

        def wait_store(c, b):
            pltpu.make_async_copy(outs[b], out_hbm.at[hslice(c)], ssem[b]).wait()

        def compute(b):
            nv, ev, ov = nums[b], engs[b], outs[b]

            @plsc.parallel_loop(0, lanes_per_w, step=_LANES, unroll=2)
            def body(cs):
                for r in range(chunk_rows):
                    s = (r, pl.ds(cs, _LANES))
                    vals = plsc.load_gather(tab_v, [nv[s]])
                    ov[s] = ev[s] + vals

        issue_loads(0, 0)

        def pair(g, carry):
            for b in range(2):
                c = 2 * g + b

                @pl.when(c + 1 < n_chunks)
                def _():
                    issue_loads(c + 1, 1 - b)

                wait_loads(c, b)

                @pl.when(c >= 2)
                def _():
                    wait_store(c - 2, b)

                compute(b)
                pltpu.async_copy(outs[b], out_hbm.at[hslice(c)], ssem[b])
            return carry

        lax.fori_loop(0, n_chunks // 2, pair, 0)
        if n_chunks % 2:
            c = n_chunks - 1
            wait_loads(c, 0)
            wait_store(c - 2, 0)
            compute(0)
            pltpu.async_copy(outs[0], out_hbm.at[hslice(c)], ssem[0])
        wait_store(n_chunks - 1, (n_chunks - 1) % 2)
        wait_store(n_chunks - 2, (n_chunks - 2) % 2)

    return run


def kernel(numbers, energy, shifts_weight):
    b, l = energy.shape
    tab_flat = shifts_weight.reshape(-1)
    out_t = _make_kernel(l, b, tab_flat.shape[0])(
        numbers.astype(jnp.int32).T, energy.T, tab_flat)
    return out_t.T
